# baseline (speedup 1.0000x reference)
import functools

import jax
import jax.numpy as jnp
from jax import lax
from jax.experimental import pallas as pl
from jax.experimental.pallas import tpu as pltpu
from jax.experimental.pallas import tpu_sc as plsc

N = 10000
E = 320000
D = 128
LANES = 16
KREG = D // LANES
NC = 2
NS = 16
NW = NC * NS
EPW = E // NW
CHUNK = 128
NFULL = EPW // CHUNK
TAIL = EPW - NFULL * CHUNK
T3 = NFULL // 3
STRIPE = 640
LAST_STRIPE = N - (NS - 1) * STRIPE

_MESH = plsc.VectorSubcoreMesh(
    core_axis_name="c", subcore_axis_name="s", num_cores=NC, num_subcores=NS)


def _zf32():
  return jnp.zeros((LANES,), jnp.float32)


@functools.partial(
    pl.kernel,
    out_type=(jax.ShapeDtypeStruct((N,), jnp.float32),
              jax.ShapeDtypeStruct((N,), jnp.float32)),
    mesh=_MESH,
    scratch_types=[
        pltpu.VMEM((EPW,), jnp.int32),
        pltpu.VMEM((EPW,), jnp.float32),
        pltpu.VMEM((2, CHUNK), jnp.int32),
        pltpu.VMEM((1, TAIL), jnp.int32),
        pltpu.VMEM((N,), jnp.float32),
        pltpu.VMEM_SHARED((N,), jnp.float32),
        pltpu.SemaphoreType.DMA,
        pltpu.SemaphoreType.DMA,
    ],
)
def _colsum(col_hbm, adj_hbm, out0_hbm, out1_hbm,
            col_v, adj_v, cidx_v, ctail_v, zb_v, cs_sh, c0, c1):
  cid = lax.axis_index("c")
  sid = lax.axis_index("s")
  wid = sid * NC + cid
  base = pl.multiple_of(wid * EPW, 8)

  pltpu.sync_copy(col_hbm.at[pl.ds(base, EPW)], col_v)
  pltpu.sync_copy(adj_hbm.at[pl.ds(base, EPW)], adj_v)

  @pl.when(sid == 0)
  def _():
    def zb_body(j, carry):
      zb_v[pl.ds(j * LANES, LANES)] = _zf32()
      return carry
    lax.fori_loop(0, N // LANES, zb_body, 0)
    pltpu.sync_copy(zb_v, cs_sh)

  plsc.subcore_barrier()

  csems = (c0, c1)

  def cfill(c, p):
    for k in range(KREG):
      cidx_v[p, pl.ds(k * LANES, LANES)] = col_v[pl.ds(c * CHUNK + k * LANES,
                                                       LANES)]

  def cfire(c, p):
    cfill(c, p)
    pltpu.async_copy(adj_v.at[pl.ds(c * CHUNK, CHUNK)],
                     cs_sh.at[cidx_v.at[p]], csems[p], add=True)

  def cwait(c, p):
    pltpu.make_async_copy(adj_v.at[pl.ds(c * CHUNK, CHUNK)],
                          cs_sh.at[cidx_v.at[p]], csems[p]).wait()

  def pair_body(t, carry):
    for u in range(2):
      c = t * 2 + u

      @pl.when(t > 0)
      def _():
        cwait(c, u)
      cfire(c, u)
    return carry

  lax.fori_loop(0, NFULL // 2, pair_body, 0)

  cwait(0, 0)
  cwait(1, 1)
  for k in range(TAIL // LANES):
    ctail_v[0, pl.ds(k * LANES, LANES)] = col_v[pl.ds(NFULL * CHUNK + k *
                                                      LANES, LANES)]
  pltpu.async_copy(adj_v.at[pl.ds(NFULL * CHUNK, TAIL)],
                   cs_sh.at[ctail_v.at[0]], c0, add=True).wait()

  plsc.subcore_barrier()

  @pl.when(sid == 0)
  def _():
    @pl.when(cid == 0)
    def _():
      pltpu.sync_copy(cs_sh, out0_hbm)

    @pl.when(cid == 1)
    def _():
      pltpu.sync_copy(cs_sh, out1_hbm)


@functools.partial(
    pl.kernel,
    out_type=jax.ShapeDtypeStruct((NC, N, D), jnp.float32),
    mesh=_MESH,
    scratch_types=[
        pltpu.VMEM((4, CHUNK), jnp.int32),
        pltpu.VMEM((4, CHUNK), jnp.int32),
        pltpu.VMEM((4, CHUNK), jnp.float32),
        pltpu.VMEM((1, TAIL), jnp.int32),
        pltpu.VMEM((1, TAIL), jnp.int32),
        pltpu.VMEM((1, TAIL), jnp.float32),
        pltpu.VMEM((CHUNK, D), jnp.float32),
        pltpu.VMEM((CHUNK, D), jnp.float32),
        pltpu.VMEM_SHARED((N, D), jnp.float32),
        pltpu.SemaphoreType.DMA,
        pltpu.SemaphoreType.DMA,
        pltpu.SemaphoreType.DMA,
        pltpu.SemaphoreType.DMA,
        pltpu.SemaphoreType.DMA,
        pltpu.SemaphoreType.DMA,
        pltpu.SemaphoreType.DMA,
        pltpu.SemaphoreType.DMA,
        pltpu.SemaphoreType.DMA,
        pltpu.SemaphoreType.DMA,
    ],
)
def _spmm(g_hbm, row_hbm, col_hbm, adj_hbm, out_hbm,
          colidx_v, rowidx_v, adjv_v, ctail_v, rtail_v, atail_v,
          rows0_v, rows1_v, acc_sh,
          g0, g1, s0, s1, s2, s3, t0, t1, t2, t3):
  cid = lax.axis_index("c")
  sid = lax.axis_index("s")
  wid = sid * NC + cid
  base = pl.multiple_of(wid * EPW, 8)

  bufs = (rows0_v, rows1_v)
  gsems = (g0, g1)
  ssems = (s0, s1, s2, s3)
  tsems = (t0, t1, t2, t3)

  def zrow_body(j, carry):
    for k in range(KREG):
      rows0_v[j, pl.ds(k * LANES, LANES)] = _zf32()
    return carry
  lax.fori_loop(0, CHUNK, zrow_body, 0)

  r0 = pl.multiple_of(sid * STRIPE, 8)

  @pl.when(sid < NS - 1)
  def _():
    for i in range(STRIPE // CHUNK):
      pltpu.sync_copy(rows0_v, acc_sh.at[pl.ds(r0 + i * CHUNK, CHUNK)])

  @pl.when(sid == NS - 1)
  def _():
    for i in range(LAST_STRIPE // CHUNK):
      pltpu.sync_copy(rows0_v, acc_sh.at[pl.ds(r0 + i * CHUNK, CHUNK)])
    rem = LAST_STRIPE - (LAST_STRIPE // CHUNK) * CHUNK
    if rem:
      pltpu.sync_copy(rows0_v.at[pl.ds(0, rem)],
                      acc_sh.at[pl.ds(r0 + LAST_STRIPE - rem, rem)])

  plsc.subcore_barrier()

  def fire_stage(c, sl):
    off = pl.multiple_of(base + c * CHUNK, 8)
    pltpu.async_copy(col_hbm.at[pl.ds(off, CHUNK)], colidx_v.at[sl],
                     tsems[sl])
    pltpu.async_copy(row_hbm.at[pl.ds(off, CHUNK)], rowidx_v.at[sl],
                     tsems[sl])
    pltpu.async_copy(adj_hbm.at[pl.ds(off, CHUNK)], adjv_v.at[sl], tsems[sl])

  def wait_stage(sl):
    off = pl.multiple_of(base, 8)
    pltpu.make_async_copy(col_hbm.at[pl.ds(off, CHUNK)], colidx_v.at[sl],
                          tsems[sl]).wait()
    pltpu.make_async_copy(row_hbm.at[pl.ds(off, CHUNK)], rowidx_v.at[sl],
                          tsems[sl]).wait()
    pltpu.make_async_copy(adj_hbm.at[pl.ds(off, CHUNK)], adjv_v.at[sl],
                          tsems[sl]).wait()

  def fire_gather(sl, p):
    pltpu.async_copy(g_hbm.at[colidx_v.at[sl]], bufs[p], gsems[p])

  def wait_gather(sl, p):
    pltpu.make_async_copy(g_hbm.at[colidx_v.at[sl]], bufs[p],
                          gsems[p]).wait()

  def fire_scatter(sl, p):
    pltpu.async_copy(bufs[p], acc_sh.at[rowidx_v.at[sl]], ssems[sl],
                     add=True, priority=1)

  def wait_scatter(sl, p):
    pltpu.make_async_copy(bufs[p], acc_sh.at[rowidx_v.at[sl]],
                          ssems[sl]).wait()

  def scale(sl, p):
    buf = bufs[p]

    def grp_body(jg, carry):
      val16 = adjv_v[sl, pl.ds(jg * LANES, LANES)]
      for jj in range(LANES):
        j = jg * LANES + jj
        val = val16[jj]
        for k in range(KREG):
          ksl = pl.ds(k * LANES, LANES)
          buf[j, ksl] = buf[j, ksl] * val
      return carry
    lax.fori_loop(0, CHUNK // LANES, grp_body, 0)

  fire_stage(0, 0)
  fire_stage(1, 1)
  fire_stage(2, 2)
  wait_stage(0)
  fire_gather(0, 0)

  def quad_body(t, carry):
    for u in range(4):
      c4 = t * 4 + u
      wait_gather(u, u % 2)
      scale(u, u % 2)
      fire_scatter(u, u % 2)
      if u == 0:
        @pl.when(t > 0)
        def _():
          wait_scatter(3, 1)
      else:
        wait_scatter(u - 1, (u - 1) % 2)
      wait_stage((u + 1) % 4)
      fire_gather((u + 1) % 4, (u + 1) % 2)

      @pl.when(c4 + 3 < NFULL)
      def _():
        fire_stage(c4 + 3, (u + 3) % 4)
    return carry

  lax.fori_loop(0, NFULL // 4, quad_body, 0)

  c = NFULL - 2
  wait_gather(0, 0)
  scale(0, 0)
  fire_scatter(0, 0)
  wait_scatter(3, 1)
  wait_stage(1)
  fire_gather(1, 1)
  wait_gather(1, 1)
  scale(1, 1)
  fire_scatter(1, 1)
  wait_scatter(0, 0)
  wait_scatter(1, 1)

  toff = pl.multiple_of(base + NFULL * CHUNK, 8)
  pltpu.sync_copy(col_hbm.at[pl.ds(toff, TAIL)], ctail_v.at[0])
  pltpu.sync_copy(row_hbm.at[pl.ds(toff, TAIL)], rtail_v.at[0])
  pltpu.sync_copy(adj_hbm.at[pl.ds(toff, TAIL)], atail_v.at[0])
  pltpu.async_copy(
      g_hbm.at[ctail_v.at[0]], rows0_v.at[pl.ds(0, TAIL)], g0).wait()
  val16 = atail_v[0, pl.ds(0, LANES)]
  for jj in range(TAIL):
    val = val16[jj]
    for k in range(KREG):
      ksl = pl.ds(k * LANES, LANES)
      rows0_v[jj, ksl] = rows0_v[jj, ksl] * val
  pltpu.async_copy(
      rows0_v.at[pl.ds(0, TAIL)], acc_sh.at[rtail_v.at[0]], s0,
      add=True).wait()

  plsc.subcore_barrier()

  @pl.when(sid < NS - 1)
  def _():
    pltpu.sync_copy(acc_sh.at[pl.ds(r0, STRIPE)],
                    out_hbm.at[cid, pl.ds(r0, STRIPE)])

  @pl.when(sid == NS - 1)
  def _():
    pltpu.sync_copy(acc_sh.at[pl.ds(r0, LAST_STRIPE)],
                    out_hbm.at[cid, pl.ds(r0, LAST_STRIPE)])


def _linw(h, w):
  return lax.dot_general(h, w, (((1,), (1,)), ((), ())),
                         preferred_element_type=jnp.float32)


_RB = 1000
_GRID = N // _RB

_vec_spec = pl.BlockSpec((_RB, D), lambda i: (i, 0))
_cs_spec = pl.BlockSpec((_RB, 1), lambda i: (i, 0))
_w_spec = pl.BlockSpec((D, D), lambda i: (0, 0))
_p_spec = pl.BlockSpec((NC, _RB, D), lambda i: (0, i, 0))


def _lin_scale_body(h_ref, w_ref, cs0_ref, cs1_ref, g_ref):
  inv = 1.0 / (cs0_ref[...] + cs1_ref[...])
  g_ref[...] = _linw(h_ref[...], w_ref[...]) * inv


_lin_scale = pl.pallas_call(
    _lin_scale_body,
    grid=(_GRID,),
    in_specs=[_vec_spec, _w_spec, _cs_spec, _cs_spec],
    out_specs=_vec_spec,
    out_shape=jax.ShapeDtypeStruct((N, D), jnp.float32),
)


def _mid_body(p_ref, acc_ref, w_ref, cs0_ref, cs1_ref, acc_out_ref, g_ref):
  s = p_ref[0] + p_ref[1]
  nrm = jnp.sqrt(jnp.sum(s * s, axis=1, keepdims=True))
  acc_out_ref[...] = acc_ref[...] + s / jnp.maximum(nrm, 1e-12)
  inv = 1.0 / (cs0_ref[...] + cs1_ref[...])
  g_ref[...] = _linw(s, w_ref[...]) * inv


_mid = pl.pallas_call(
    _mid_body,
    grid=(_GRID,),
    in_specs=[_p_spec, _vec_spec, _w_spec, _cs_spec, _cs_spec],
    out_specs=[_vec_spec, _vec_spec],
    out_shape=[
        jax.ShapeDtypeStruct((N, D), jnp.float32),
        jax.ShapeDtypeStruct((N, D), jnp.float32),
    ],
)


def _end_body(q_ref, acc_ref, out_ref):
  s = q_ref[0] + q_ref[1]
  nrm = jnp.sqrt(jnp.sum(s * s, axis=1, keepdims=True))
  out_ref[...] = (acc_ref[...] + s / jnp.maximum(nrm, 1e-12)) * (1.0 / 3.0)


_end = pl.pallas_call(
    _end_body,
    grid=(_GRID,),
    in_specs=[_p_spec, _vec_spec],
    out_specs=_vec_spec,
    out_shape=jax.ShapeDtypeStruct((N, D), jnp.float32),
)


def kernel(embedding, edge_index, adj_values, W0, W1):
  row = edge_index[0].astype(jnp.int32)
  col = edge_index[1].astype(jnp.int32)
  adj = adj_values

  cs0, cs1 = _colsum(col, adj)
  cs0 = cs0.reshape(N, 1)
  cs1 = cs1.reshape(N, 1)

  g1 = _lin_scale(embedding, W0, cs0, cs1)
  p = _spmm(g1, row, col, adj)
  acc1, g2 = _mid(p, embedding, W1, cs0, cs1)
  q = _spmm(g2, row, col, adj)
  return _end(q, acc1)


# speedup vs baseline: 1.0182x; 1.0182x over previous
import functools

import jax
import jax.numpy as jnp
from jax import lax
from jax.experimental import pallas as pl
from jax.experimental.pallas import tpu as pltpu
from jax.experimental.pallas import tpu_sc as plsc

N = 10000
E = 320000
D = 128
LANES = 16
KREG = D // LANES
NC = 2
NS = 16
NW = NC * NS
EPW = E // NW
CHUNK = 128
NFULL = EPW // CHUNK
TAIL = EPW - NFULL * CHUNK
T3 = NFULL // 3
STRIPE = 640
LAST_STRIPE = N - (NS - 1) * STRIPE

_MESH = plsc.VectorSubcoreMesh(
    core_axis_name="c", subcore_axis_name="s", num_cores=NC, num_subcores=NS)


def _zf32():
  return jnp.zeros((LANES,), jnp.float32)


@functools.partial(
    pl.kernel,
    out_type=(jax.ShapeDtypeStruct((N,), jnp.float32),
              jax.ShapeDtypeStruct((N,), jnp.float32)),
    mesh=_MESH,
    scratch_types=[
        pltpu.VMEM((EPW,), jnp.int32),
        pltpu.VMEM((EPW,), jnp.float32),
        pltpu.VMEM((4, CHUNK), jnp.int32),
        pltpu.VMEM((1, TAIL), jnp.int32),
        pltpu.VMEM((N,), jnp.float32),
        pltpu.VMEM_SHARED((N,), jnp.float32),
        pltpu.SemaphoreType.DMA,
        pltpu.SemaphoreType.DMA,
        pltpu.SemaphoreType.DMA,
        pltpu.SemaphoreType.DMA,
    ],
)
def _colsum(col_hbm, adj_hbm, out0_hbm, out1_hbm,
            col_v, adj_v, cidx_v, ctail_v, zb_v, cs_sh, c0, c1, c2, c3):
  cid = lax.axis_index("c")
  sid = lax.axis_index("s")
  wid = sid * NC + cid
  base = pl.multiple_of(wid * EPW, 8)

  pltpu.sync_copy(col_hbm.at[pl.ds(base, EPW)], col_v)
  pltpu.sync_copy(adj_hbm.at[pl.ds(base, EPW)], adj_v)

  @pl.when(sid == 0)
  def _():
    def zb_body(j, carry):
      zb_v[pl.ds(j * LANES, LANES)] = _zf32()
      return carry
    lax.fori_loop(0, N // LANES, zb_body, 0)
    pltpu.sync_copy(zb_v, cs_sh)

  plsc.subcore_barrier()

  csems = (c0, c1, c2, c3)

  def cfill(c, p):
    for k in range(KREG):
      cidx_v[p, pl.ds(k * LANES, LANES)] = col_v[pl.ds(c * CHUNK + k * LANES,
                                                       LANES)]

  def cfire(c, p):
    cfill(c, p)
    pltpu.async_copy(adj_v.at[pl.ds(c * CHUNK, CHUNK)],
                     cs_sh.at[cidx_v.at[p]], csems[p], add=True)

  def cwait(c, p):
    pltpu.make_async_copy(adj_v.at[pl.ds(c * CHUNK, CHUNK)],
                          cs_sh.at[cidx_v.at[p]], csems[p]).wait()

  def quad_body(t, carry):
    for u in range(4):
      c = t * 4 + u

      @pl.when(t > 0)
      def _():
        cwait(c, u)
      cfire(c, u)
    return carry

  lax.fori_loop(0, NFULL // 4, quad_body, 0)

  cwait(0, 0)
  cfire(NFULL - 2, 0)
  cwait(1, 1)
  cfire(NFULL - 1, 1)
  cwait(2, 2)
  for k in range(TAIL // LANES):
    ctail_v[0, pl.ds(k * LANES, LANES)] = col_v[pl.ds(NFULL * CHUNK + k *
                                                      LANES, LANES)]
  pltpu.async_copy(adj_v.at[pl.ds(NFULL * CHUNK, TAIL)],
                   cs_sh.at[ctail_v.at[0]], c2, add=True).wait()
  cwait(3, 3)
  cwait(0, 0)
  cwait(1, 1)

  plsc.subcore_barrier()

  @pl.when(sid == 0)
  def _():
    @pl.when(cid == 0)
    def _():
      pltpu.sync_copy(cs_sh, out0_hbm)

    @pl.when(cid == 1)
    def _():
      pltpu.sync_copy(cs_sh, out1_hbm)


@functools.partial(
    pl.kernel,
    out_type=jax.ShapeDtypeStruct((NC, N, D), jnp.float32),
    mesh=_MESH,
    scratch_types=[
        pltpu.VMEM((4, CHUNK), jnp.int32),
        pltpu.VMEM((4, CHUNK), jnp.int32),
        pltpu.VMEM((4, CHUNK), jnp.float32),
        pltpu.VMEM((1, TAIL), jnp.int32),
        pltpu.VMEM((1, TAIL), jnp.int32),
        pltpu.VMEM((1, TAIL), jnp.float32),
        pltpu.VMEM((CHUNK, D), jnp.float32),
        pltpu.VMEM((CHUNK, D), jnp.float32),
        pltpu.VMEM_SHARED((N, D), jnp.float32),
        pltpu.SemaphoreType.DMA,
        pltpu.SemaphoreType.DMA,
        pltpu.SemaphoreType.DMA,
        pltpu.SemaphoreType.DMA,
        pltpu.SemaphoreType.DMA,
        pltpu.SemaphoreType.DMA,
        pltpu.SemaphoreType.DMA,
        pltpu.SemaphoreType.DMA,
        pltpu.SemaphoreType.DMA,
        pltpu.SemaphoreType.DMA,
    ],
)
def _spmm(g_hbm, row_hbm, col_hbm, adj_hbm, out_hbm,
          colidx_v, rowidx_v, adjv_v, ctail_v, rtail_v, atail_v,
          rows0_v, rows1_v, acc_sh,
          g0, g1, s0, s1, s2, s3, t0, t1, t2, t3):
  cid = lax.axis_index("c")
  sid = lax.axis_index("s")
  wid = sid * NC + cid
  base = pl.multiple_of(wid * EPW, 8)

  bufs = (rows0_v, rows1_v)
  gsems = (g0, g1)
  ssems = (s0, s1, s2, s3)
  tsems = (t0, t1, t2, t3)

  def zrow_body(j, carry):
    for k in range(KREG):
      rows0_v[j, pl.ds(k * LANES, LANES)] = _zf32()
    return carry
  lax.fori_loop(0, CHUNK, zrow_body, 0)

  r0 = pl.multiple_of(sid * STRIPE, 8)

  @pl.when(sid < NS - 1)
  def _():
    for i in range(STRIPE // CHUNK):
      pltpu.sync_copy(rows0_v, acc_sh.at[pl.ds(r0 + i * CHUNK, CHUNK)])

  @pl.when(sid == NS - 1)
  def _():
    for i in range(LAST_STRIPE // CHUNK):
      pltpu.sync_copy(rows0_v, acc_sh.at[pl.ds(r0 + i * CHUNK, CHUNK)])
    rem = LAST_STRIPE - (LAST_STRIPE // CHUNK) * CHUNK
    if rem:
      pltpu.sync_copy(rows0_v.at[pl.ds(0, rem)],
                      acc_sh.at[pl.ds(r0 + LAST_STRIPE - rem, rem)])

  plsc.subcore_barrier()

  def fire_stage(c, sl):
    off = pl.multiple_of(base + c * CHUNK, 8)
    pltpu.async_copy(col_hbm.at[pl.ds(off, CHUNK)], colidx_v.at[sl],
                     tsems[sl])
    pltpu.async_copy(row_hbm.at[pl.ds(off, CHUNK)], rowidx_v.at[sl],
                     tsems[sl])
    pltpu.async_copy(adj_hbm.at[pl.ds(off, CHUNK)], adjv_v.at[sl], tsems[sl])

  def wait_stage(sl):
    off = pl.multiple_of(base, 8)
    pltpu.make_async_copy(col_hbm.at[pl.ds(off, CHUNK)], colidx_v.at[sl],
                          tsems[sl]).wait()
    pltpu.make_async_copy(row_hbm.at[pl.ds(off, CHUNK)], rowidx_v.at[sl],
                          tsems[sl]).wait()
    pltpu.make_async_copy(adj_hbm.at[pl.ds(off, CHUNK)], adjv_v.at[sl],
                          tsems[sl]).wait()

  def fire_gather(sl, p):
    pltpu.async_copy(g_hbm.at[colidx_v.at[sl]], bufs[p], gsems[p])

  def wait_gather(sl, p):
    pltpu.make_async_copy(g_hbm.at[colidx_v.at[sl]], bufs[p],
                          gsems[p]).wait()

  def fire_scatter(sl, p):
    pltpu.async_copy(bufs[p], acc_sh.at[rowidx_v.at[sl]], ssems[sl],
                     add=True)

  def wait_scatter(sl, p):
    pltpu.make_async_copy(bufs[p], acc_sh.at[rowidx_v.at[sl]],
                          ssems[sl]).wait()

  def scale(sl, p):
    buf = bufs[p]

    def grp_body(jg, carry):
      val16 = adjv_v[sl, pl.ds(jg * LANES, LANES)]
      for jj in range(LANES):
        j = jg * LANES + jj
        val = val16[jj]
        for k in range(KREG):
          ksl = pl.ds(k * LANES, LANES)
          buf[j, ksl] = buf[j, ksl] * val
      return carry
    lax.fori_loop(0, CHUNK // LANES, grp_body, 0)

  fire_stage(0, 0)
  fire_stage(1, 1)
  fire_stage(2, 2)
  wait_stage(0)
  fire_gather(0, 0)

  def quad_body(t, carry):
    for u in range(4):
      c4 = t * 4 + u
      wait_gather(u, u % 2)
      scale(u, u % 2)
      fire_scatter(u, u % 2)
      if u == 0:
        @pl.when(t > 0)
        def _():
          wait_scatter(3, 1)
      else:
        wait_scatter(u - 1, (u - 1) % 2)
      wait_stage((u + 1) % 4)
      fire_gather((u + 1) % 4, (u + 1) % 2)

      @pl.when(c4 + 3 < NFULL)
      def _():
        fire_stage(c4 + 3, (u + 3) % 4)
    return carry

  lax.fori_loop(0, NFULL // 4, quad_body, 0)

  c = NFULL - 2
  wait_gather(0, 0)
  scale(0, 0)
  fire_scatter(0, 0)
  wait_scatter(3, 1)
  wait_stage(1)
  fire_gather(1, 1)
  wait_gather(1, 1)
  scale(1, 1)
  fire_scatter(1, 1)
  wait_scatter(0, 0)
  wait_scatter(1, 1)

  toff = pl.multiple_of(base + NFULL * CHUNK, 8)
  pltpu.sync_copy(col_hbm.at[pl.ds(toff, TAIL)], ctail_v.at[0])
  pltpu.sync_copy(row_hbm.at[pl.ds(toff, TAIL)], rtail_v.at[0])
  pltpu.sync_copy(adj_hbm.at[pl.ds(toff, TAIL)], atail_v.at[0])
  pltpu.async_copy(
      g_hbm.at[ctail_v.at[0]], rows0_v.at[pl.ds(0, TAIL)], g0).wait()
  val16 = atail_v[0, pl.ds(0, LANES)]
  for jj in range(TAIL):
    val = val16[jj]
    for k in range(KREG):
      ksl = pl.ds(k * LANES, LANES)
      rows0_v[jj, ksl] = rows0_v[jj, ksl] * val
  pltpu.async_copy(
      rows0_v.at[pl.ds(0, TAIL)], acc_sh.at[rtail_v.at[0]], s0,
      add=True).wait()

  plsc.subcore_barrier()

  @pl.when(sid < NS - 1)
  def _():
    pltpu.sync_copy(acc_sh.at[pl.ds(r0, STRIPE)],
                    out_hbm.at[cid, pl.ds(r0, STRIPE)])

  @pl.when(sid == NS - 1)
  def _():
    pltpu.sync_copy(acc_sh.at[pl.ds(r0, LAST_STRIPE)],
                    out_hbm.at[cid, pl.ds(r0, LAST_STRIPE)])


def _linw(h, w):
  return lax.dot_general(h, w, (((1,), (1,)), ((), ())),
                         preferred_element_type=jnp.float32)


def _lin_scale_body(h_ref, w_ref, cs0_ref, cs1_ref, g_ref):
  inv = 1.0 / (cs0_ref[...] + cs1_ref[...])
  g_ref[...] = _linw(h_ref[...], w_ref[...]) * inv


_lin_scale = pl.pallas_call(
    _lin_scale_body,
    out_shape=jax.ShapeDtypeStruct((N, D), jnp.float32),
)


def _mid_body(p_ref, acc_ref, w_ref, cs0_ref, cs1_ref, acc_out_ref, g_ref):
  s = p_ref[0] + p_ref[1]
  nrm = jnp.sqrt(jnp.sum(s * s, axis=1, keepdims=True))
  acc_out_ref[...] = acc_ref[...] + s / jnp.maximum(nrm, 1e-12)
  inv = 1.0 / (cs0_ref[...] + cs1_ref[...])
  g_ref[...] = _linw(s, w_ref[...]) * inv


_mid = pl.pallas_call(
    _mid_body,
    out_shape=[
        jax.ShapeDtypeStruct((N, D), jnp.float32),
        jax.ShapeDtypeStruct((N, D), jnp.float32),
    ],
)


def _end_body(q_ref, acc_ref, out_ref):
  s = q_ref[0] + q_ref[1]
  nrm = jnp.sqrt(jnp.sum(s * s, axis=1, keepdims=True))
  out_ref[...] = (acc_ref[...] + s / jnp.maximum(nrm, 1e-12)) * (1.0 / 3.0)


_end = pl.pallas_call(
    _end_body,
    out_shape=jax.ShapeDtypeStruct((N, D), jnp.float32),
)


def kernel(embedding, edge_index, adj_values, W0, W1):
  row = edge_index[0].astype(jnp.int32)
  col = edge_index[1].astype(jnp.int32)
  adj = adj_values

  cs0, cs1 = _colsum(col, adj)
  cs0 = cs0.reshape(N, 1)
  cs1 = cs1.reshape(N, 1)

  g1 = _lin_scale(embedding, W0, cs0, cs1)
  p = _spmm(g1, row, col, adj)
  acc1, g2 = _mid(p, embedding, W1, cs0, cs1)
  q = _spmm(g2, row, col, adj)
  return _end(q, acc1)
